# baseline (speedup 1.0000x reference)
import functools

import jax
import jax.numpy as jnp
from jax import lax
from jax.experimental import pallas as pl
from jax.experimental.pallas import tpu as pltpu
from jax.experimental.pallas import tpu_sc as plsc

N = 10000
E = 320000
NP = 10240
D = 128
BLK = 2048
NT = 32
DPT = D // NT
EC = 8000
EPT = E // NT


def _feat_segsum_body(yt_hbm, src_hbm, dst_hbm, out_hbm, tab, acc,
                      sbuf0, dbuf0, sbuf1, dbuf1,
                      sem_t, sem_s0, sem_d0, sem_s1, sem_d1):
    wid = lax.axis_index("s") * 2 + lax.axis_index("c")
    r0 = wid * DPT
    tab_cp = pltpu.async_copy(yt_hbm.at[pl.ds(r0, DPT)], tab, sem_t)
    pltpu.async_copy(src_hbm.at[pl.ds(0, EC)], sbuf0, sem_s0)
    pltpu.async_copy(dst_hbm.at[pl.ds(0, EC)], dbuf0, sem_d0)

    zero = jnp.zeros((16,), jnp.float32)

    @plsc.parallel_loop(0, NP // 16, unroll=8)
    def _zero(i):
        for d in range(DPT):
            acc[d, pl.ds(i * 16, 16)] = zero

    tab_cp.wait()

    dvecs = [jnp.full((16,), d, jnp.int32) for d in range(DPT)]

    def _process(sb, db):
        @plsc.parallel_loop(0, EC // 16, unroll=8)
        def _groups(g):
            sv = sb[pl.ds(g * 16, 16)]
            dv = db[pl.ds(g * 16, 16)]
            for d in range(DPT):
                vals = plsc.load_gather(tab, [dvecs[d], sv])
                plsc.addupdate_scatter(acc, [dvecs[d], dv], vals)

    def cbody(i, carry):
        c0 = 2 * i
        c1 = 2 * i + 1
        pltpu.async_copy(src_hbm.at[pl.ds(c1 * EC, EC)], sbuf1, sem_s1)
        pltpu.async_copy(dst_hbm.at[pl.ds(c1 * EC, EC)], dbuf1, sem_d1)
        pltpu.make_async_copy(src_hbm.at[pl.ds(c0 * EC, EC)], sbuf0, sem_s0).wait()
        pltpu.make_async_copy(dst_hbm.at[pl.ds(c0 * EC, EC)], dbuf0, sem_d0).wait()
        _process(sbuf0, dbuf0)
        nxt = c0 + 2

        @pl.when(nxt < E // EC)
        def _():
            pltpu.async_copy(src_hbm.at[pl.ds(nxt * EC, EC)], sbuf0, sem_s0)
            pltpu.async_copy(dst_hbm.at[pl.ds(nxt * EC, EC)], dbuf0, sem_d0)

        pltpu.make_async_copy(src_hbm.at[pl.ds(c1 * EC, EC)], sbuf1, sem_s1).wait()
        pltpu.make_async_copy(dst_hbm.at[pl.ds(c1 * EC, EC)], dbuf1, sem_d1).wait()
        _process(sbuf1, dbuf1)
        return carry

    lax.fori_loop(0, E // EC // 2, cbody, 0)
    pltpu.sync_copy(acc, out_hbm.at[pl.ds(r0, DPT)])


def _scalar_segsum_body(tab_hbm, src_hbm, dst_hbm, out_hbm, tab, acc, sbuf, dbuf):
    wid = lax.axis_index("s") * 2 + lax.axis_index("c")
    base = wid * EPT
    pltpu.sync_copy(tab_hbm, tab)
    pltpu.sync_copy(src_hbm.at[pl.ds(base, EPT)], sbuf)
    pltpu.sync_copy(dst_hbm.at[pl.ds(base, EPT)], dbuf)

    zero = jnp.zeros((16,), jnp.float32)

    @plsc.parallel_loop(0, NP // 16, unroll=8)
    def _zero(i):
        acc[pl.ds(i * 16, 16)] = zero

    @plsc.parallel_loop(0, EPT // 16, unroll=8)
    def _groups(g):
        sv = sbuf[pl.ds(g * 16, 16)]
        dv = dbuf[pl.ds(g * 16, 16)]
        vals = plsc.load_gather(tab, [sv])
        plsc.addupdate_scatter(acc, [dv], vals)

    pltpu.sync_copy(acc, out_hbm.at[wid])


@functools.cache
def _sc_kernels():
    mesh = plsc.VectorSubcoreMesh(core_axis_name="c", subcore_axis_name="s")
    params = pltpu.CompilerParams(needs_layout_passes=False)
    feat = pl.kernel(
        _feat_segsum_body,
        mesh=mesh,
        compiler_params=params,
        out_type=jax.ShapeDtypeStruct((D, NP), jnp.float32),
        scratch_types=[
            pltpu.VMEM((DPT, NP), jnp.float32),
            pltpu.VMEM((DPT, NP), jnp.float32),
            pltpu.VMEM((EC,), jnp.int32),
            pltpu.VMEM((EC,), jnp.int32),
            pltpu.VMEM((EC,), jnp.int32),
            pltpu.VMEM((EC,), jnp.int32),
            pltpu.SemaphoreType.DMA,
            pltpu.SemaphoreType.DMA,
            pltpu.SemaphoreType.DMA,
            pltpu.SemaphoreType.DMA,
            pltpu.SemaphoreType.DMA,
        ],
    )
    scalar = pl.kernel(
        _scalar_segsum_body,
        mesh=mesh,
        compiler_params=params,
        out_type=jax.ShapeDtypeStruct((NT, NP), jnp.float32),
        scratch_types=[
            pltpu.VMEM((NP,), jnp.float32),
            pltpu.VMEM((NP,), jnp.float32),
            pltpu.VMEM((EPT,), jnp.int32),
            pltpu.VMEM((EPT,), jnp.int32),
        ],
    )
    return feat, scalar


_DN_T = (((1,), (1,)), ((), ()))
_DN = (((1,), (0,)), ((), ()))


def _l1_body(x_ref, wl_ref, wr_ref, b_ref, y_ref, z_ref):
    xb = x_ref[...]
    y_ref[...] = lax.dot_general(wl_ref[...], xb, _DN_T,
                                 preferred_element_type=jnp.float32)
    z_ref[...] = lax.dot_general(wr_ref[...], xb, _DN_T,
                                 preferred_element_type=jnp.float32) + b_ref[...]


def _mid_body(agg_ref, z_ref, dinv_ref, a_ref, wl_ref, wr_ref, b_ref,
              y_ref, zo_ref):
    pre = agg_ref[...] * dinv_ref[...] + z_ref[...]
    h = jnp.where(pre >= 0, pre, a_ref[...] * pre)
    y_ref[...] = lax.dot_general(wl_ref[...], h, _DN,
                                 preferred_element_type=jnp.float32)
    zo_ref[...] = lax.dot_general(wr_ref[...], h, _DN,
                                  preferred_element_type=jnp.float32) + b_ref[...]


def _head_body(agg_ref, z_ref, dinv_ref, a_ref, uv_ref, st_ref):
    pre = agg_ref[...] * dinv_ref[...] + z_ref[...]
    h = jnp.where(pre >= 0, pre, a_ref[...] * pre)
    st_ref[...] = lax.dot_general(uv_ref[...], h, _DN,
                                  preferred_element_type=jnp.float32)


def _dinv_body(cnt_ref, dinv_ref):
    s = jnp.sum(cnt_ref[...], axis=0, keepdims=True)
    dinv_ref[...] = 1.0 / jnp.maximum(s, 1.0)


def _final_body(part_ref, st_ref, dinv_ref, c_ref, out_ref):
    s = jnp.sum(part_ref[...], axis=0, keepdims=True)
    out_ref[...] = s * dinv_ref[...] + st_ref[1:2, :] + c_ref[...]


def _full(shape):
    return pl.BlockSpec(shape, lambda j: (0,) * len(shape))


def _tc_l1(x, wl, wr, b):
    return pl.pallas_call(
        _l1_body,
        grid=(NP // BLK,),
        in_specs=[pl.BlockSpec((BLK, D), lambda j: (j, 0)),
                  _full((D, D)), _full((D, D)), _full((D, 1))],
        out_specs=[pl.BlockSpec((D, BLK), lambda j: (0, j)),
                   pl.BlockSpec((D, BLK), lambda j: (0, j))],
        out_shape=[jax.ShapeDtypeStruct((D, NP), jnp.float32)] * 2,
    )(x, wl, wr, b)


def _tc_mid(agg, z, dinv, a, wl, wr, b):
    return pl.pallas_call(
        _mid_body,
        grid=(NP // BLK,),
        in_specs=[pl.BlockSpec((D, BLK), lambda j: (0, j)),
                  pl.BlockSpec((D, BLK), lambda j: (0, j)),
                  pl.BlockSpec((1, BLK), lambda j: (0, j)),
                  _full((1, 1)), _full((D, D)), _full((D, D)), _full((D, 1))],
        out_specs=[pl.BlockSpec((D, BLK), lambda j: (0, j)),
                   pl.BlockSpec((D, BLK), lambda j: (0, j))],
        out_shape=[jax.ShapeDtypeStruct((D, NP), jnp.float32)] * 2,
    )(agg, z, dinv, a, wl, wr, b)


def _tc_head(agg, z, dinv, a, uv):
    return pl.pallas_call(
        _head_body,
        grid=(NP // BLK,),
        in_specs=[pl.BlockSpec((D, BLK), lambda j: (0, j)),
                  pl.BlockSpec((D, BLK), lambda j: (0, j)),
                  pl.BlockSpec((1, BLK), lambda j: (0, j)),
                  _full((1, 1)), _full((2, D))],
        out_specs=pl.BlockSpec((2, BLK), lambda j: (0, j)),
        out_shape=jax.ShapeDtypeStruct((2, NP), jnp.float32),
    )(agg, z, dinv, a, uv)


def _tc_dinv(cnt):
    return pl.pallas_call(
        _dinv_body,
        grid=(NP // BLK,),
        in_specs=[pl.BlockSpec((NT, BLK), lambda j: (0, j))],
        out_specs=pl.BlockSpec((1, BLK), lambda j: (0, j)),
        out_shape=jax.ShapeDtypeStruct((1, NP), jnp.float32),
    )(cnt)


def _tc_final(part, st, dinv, c):
    return pl.pallas_call(
        _final_body,
        grid=(NP // BLK,),
        in_specs=[pl.BlockSpec((NT, BLK), lambda j: (0, j)),
                  pl.BlockSpec((2, BLK), lambda j: (0, j)),
                  pl.BlockSpec((1, BLK), lambda j: (0, j)),
                  _full((1, 1))],
        out_specs=pl.BlockSpec((1, BLK), lambda j: (0, j)),
        out_shape=jax.ShapeDtypeStruct((1, NP), jnp.float32),
    )(part, st, dinv, c)


def kernel(x, edge_index, W1l, b1, W1r, W2l, b2, W2r, W3l, b3, W3r, a, Wp, bp):
    src = edge_index[0]
    dst = edge_index[1]
    xp = jnp.pad(x, ((0, NP - N), (0, 0)))
    ones = jnp.ones((NP,), jnp.float32)
    a2 = jnp.reshape(a, (1, 1))
    b1c = jnp.reshape(b1, (D, 1))
    b2c = jnp.reshape(b2, (D, 1))
    uv = jnp.concatenate([Wp @ W3l, Wp @ W3r], axis=0)
    c = jnp.reshape(Wp @ b3 + bp, (1, 1))

    feat_segsum, scalar_segsum = _sc_kernels()
    cnt = scalar_segsum(ones, src, dst)
    dinv = _tc_dinv(cnt)
    y1, z1 = _tc_l1(xp, W1l, W1r, b1c)
    agg1 = feat_segsum(y1, src, dst)
    y2, z2 = _tc_mid(agg1, z1, dinv, a2, W2l, W2r, b2c)
    agg2 = feat_segsum(y2, src, dst)
    st = _tc_head(agg2, z2, dinv, a2, uv)
    spart = scalar_segsum(st[0], src, dst)
    out = _tc_final(spart, st, dinv, c)
    return out[0, :N]


# speedup vs baseline: 1.2278x; 1.2278x over previous
import functools

import jax
import jax.numpy as jnp
from jax import lax
from jax.experimental import pallas as pl
from jax.experimental.pallas import tpu as pltpu
from jax.experimental.pallas import tpu_sc as plsc

N = 10000
E = 320000
NP = 10240
D = 128
BLK = 2048
NT = 32
NSUB = 16
K = 128
NCH = E // K
DH = D // 2
CPT = NCH // NSUB
CREM = NCH - CPT * NSUB
RPS = NP // NSUB
EPT = E // NT


def _feat_segsum_body(yab_hbm, src_hbm, dst_hbm, out_hbm,
                      acc, sidx, didx, rows0, rows1,
                      sem_g0, sem_g1):
    cid = lax.axis_index("c")
    sid = lax.axis_index("s")
    ytab = yab_hbm.at[cid]
    c0 = CPT * sid + jnp.minimum(sid, CREM)
    nch = CPT + (sid < CREM).astype(jnp.int32)

    pltpu.sync_copy(src_hbm.at[pl.ds(c0, CPT)], sidx.at[pl.ds(0, CPT)])
    pltpu.sync_copy(dst_hbm.at[pl.ds(c0, CPT)], didx.at[pl.ds(0, CPT)])

    @pl.when(sid < CREM)
    def _():
        pltpu.sync_copy(src_hbm.at[pl.ds(c0 + CPT, 1)], sidx.at[pl.ds(CPT, 1)])
        pltpu.sync_copy(dst_hbm.at[pl.ds(c0 + CPT, 1)], didx.at[pl.ds(CPT, 1)])

    zero = jnp.zeros((16,), jnp.float32)

    @plsc.parallel_loop(0, K, unroll=4)
    def _zrows(i):
        for j in range(DH // 16):
            rows0[i, pl.ds(j * 16, 16)] = zero

    for r in range(RPS // K):
        pltpu.sync_copy(rows0, acc.at[pl.ds(sid * RPS + r * K, K)])
    plsc.subcore_barrier()

    pltpu.make_async_copy(ytab.at[sidx.at[0]], rows0, sem_g0).start()

    def _step(t, buf, sem, obuf, osem):
        @pl.when(t + 1 < nch)
        def _():
            pltpu.make_async_copy(ytab.at[sidx.at[t + 1]], obuf, osem).start()

        pltpu.make_async_copy(ytab.at[sidx.at[t]], buf, sem).wait()
        pltpu.sync_copy(buf, acc.at[didx.at[t]], add=True)

    def mbody(t, carry):
        @pl.when(t % 2 == 0)
        def _():
            _step(t, rows0, sem_g0, rows1, sem_g1)

        @pl.when(t % 2 == 1)
        def _():
            _step(t, rows1, sem_g1, rows0, sem_g0)

        return carry

    lax.fori_loop(0, nch, mbody, 0)
    plsc.subcore_barrier()

    pltpu.sync_copy(acc.at[pl.ds(sid * RPS, RPS)],
                    out_hbm.at[cid, pl.ds(sid * RPS, RPS)])


def _scalar_segsum_body(tab_hbm, src_hbm, dst_hbm, out_hbm, tab, acc, sbuf, dbuf):
    wid = lax.axis_index("s") * 2 + lax.axis_index("c")
    base = wid * EPT
    pltpu.sync_copy(tab_hbm, tab)
    pltpu.sync_copy(src_hbm.at[pl.ds(base, EPT)], sbuf)
    pltpu.sync_copy(dst_hbm.at[pl.ds(base, EPT)], dbuf)

    zero = jnp.zeros((16,), jnp.float32)

    @plsc.parallel_loop(0, NP // 16, unroll=8)
    def _zero(i):
        acc[pl.ds(i * 16, 16)] = zero

    @plsc.parallel_loop(0, EPT // 16, unroll=8)
    def _groups(g):
        sv = sbuf[pl.ds(g * 16, 16)]
        dv = dbuf[pl.ds(g * 16, 16)]
        vals = plsc.load_gather(tab, [sv])
        plsc.addupdate_scatter(acc, [dv], vals)

    pltpu.sync_copy(acc, out_hbm.at[wid])


@functools.cache
def _sc_kernels():
    mesh = plsc.VectorSubcoreMesh(core_axis_name="c", subcore_axis_name="s")
    params = pltpu.CompilerParams(needs_layout_passes=False)
    feat = pl.kernel(
        _feat_segsum_body,
        mesh=mesh,
        compiler_params=pltpu.CompilerParams(
            needs_layout_passes=False, use_tc_tiling_on_sc=False),
        out_type=jax.ShapeDtypeStruct((2, NP, DH), jnp.float32),
        scratch_types=[
            pltpu.MemorySpace.VMEM_SHARED((NP, DH), jnp.float32),
            pltpu.VMEM((CPT + 1, K), jnp.int32),
            pltpu.VMEM((CPT + 1, K), jnp.int32),
            pltpu.VMEM((K, DH), jnp.float32),
            pltpu.VMEM((K, DH), jnp.float32),
            pltpu.SemaphoreType.DMA,
            pltpu.SemaphoreType.DMA,
        ],
    )
    scalar = pl.kernel(
        _scalar_segsum_body,
        mesh=mesh,
        compiler_params=params,
        out_type=jax.ShapeDtypeStruct((NT, NP), jnp.float32),
        scratch_types=[
            pltpu.VMEM((NP,), jnp.float32),
            pltpu.VMEM((NP,), jnp.float32),
            pltpu.VMEM((EPT,), jnp.int32),
            pltpu.VMEM((EPT,), jnp.int32),
        ],
    )
    return feat, scalar


_DN_T = (((1,), (1,)), ((), ()))


def _split_store(yab_ref, y):
    yab_ref[0] = y[:, :DH]
    yab_ref[1] = y[:, DH:]


def _l1_body(x_ref, wl_ref, wr_ref, b_ref, yab_ref, z_ref):
    xb = x_ref[...]
    _split_store(yab_ref, lax.dot_general(xb, wl_ref[...], _DN_T,
                                          preferred_element_type=jnp.float32))
    z_ref[...] = lax.dot_general(xb, wr_ref[...], _DN_T,
                                 preferred_element_type=jnp.float32) + b_ref[...]


def _mid_body(agg_ref, z_ref, dinv_ref, a_ref, wl_ref, wr_ref,
              b_ref, yab_ref, zo_ref):
    agg = jnp.concatenate([agg_ref[0], agg_ref[1]], axis=1)
    pre = agg * dinv_ref[...] + z_ref[...]
    h = jnp.where(pre >= 0, pre, a_ref[...] * pre)
    _split_store(yab_ref, lax.dot_general(h, wl_ref[...], _DN_T,
                                          preferred_element_type=jnp.float32))
    zo_ref[...] = lax.dot_general(h, wr_ref[...], _DN_T,
                                  preferred_element_type=jnp.float32) + b_ref[...]


def _head_body(agg_ref, z_ref, dinv_ref, a_ref, uv_ref, st_ref):
    agg = jnp.concatenate([agg_ref[0], agg_ref[1]], axis=1)
    pre = agg * dinv_ref[...] + z_ref[...]
    h = jnp.where(pre >= 0, pre, a_ref[...] * pre)
    st_ref[...] = lax.dot_general(h, uv_ref[...], _DN_T,
                                  preferred_element_type=jnp.float32)


def _dinv_body(cnt_ref, dinv_ref):
    s = jnp.sum(cnt_ref[...], axis=0, keepdims=True)
    dinv_ref[...] = 1.0 / jnp.maximum(s, 1.0)


def _final_body(part_ref, t_ref, dinv_ref, c_ref, out_ref):
    s = jnp.sum(part_ref[...], axis=0, keepdims=True)
    out_ref[...] = s * dinv_ref[...] + t_ref[...] + c_ref[...]


def _full(shape):
    return pl.BlockSpec(shape, lambda j: (0,) * len(shape))


def _nblk(shape2):
    return pl.BlockSpec(shape2, lambda j: (j, 0))


_AB_SPEC = pl.BlockSpec((2, BLK, DH), lambda j: (0, j, 0))
_AB_SHAPE = jax.ShapeDtypeStruct((2, NP, DH), jnp.float32)


def _tc_l1(x, wl, wr, b):
    return pl.pallas_call(
        _l1_body,
        grid=(NP // BLK,),
        in_specs=[_nblk((BLK, D)), _full((D, D)), _full((D, D)), _full((1, D))],
        out_specs=[_AB_SPEC, _nblk((BLK, D))],
        out_shape=[_AB_SHAPE, jax.ShapeDtypeStruct((NP, D), jnp.float32)],
    )(x, wl, wr, b)


def _tc_mid(agg, z, dinv, a, wl, wr, b):
    return pl.pallas_call(
        _mid_body,
        grid=(NP // BLK,),
        in_specs=[_AB_SPEC, _nblk((BLK, D)), _nblk((BLK, 1)),
                  _full((1, 1)), _full((D, D)), _full((D, D)), _full((1, D))],
        out_specs=[_AB_SPEC, _nblk((BLK, D))],
        out_shape=[_AB_SHAPE, jax.ShapeDtypeStruct((NP, D), jnp.float32)],
    )(agg, z, dinv, a, wl, wr, b)


def _tc_head(agg, z, dinv, a, uv):
    return pl.pallas_call(
        _head_body,
        grid=(NP // BLK,),
        in_specs=[_AB_SPEC, _nblk((BLK, D)), _nblk((BLK, 1)),
                  _full((1, 1)), _full((2, D))],
        out_specs=_nblk((BLK, 2)),
        out_shape=jax.ShapeDtypeStruct((NP, 2), jnp.float32),
    )(agg, z, dinv, a, uv)


def _tc_dinv(cnt):
    return pl.pallas_call(
        _dinv_body,
        grid=(NP // BLK,),
        in_specs=[pl.BlockSpec((NT, BLK), lambda j: (0, j))],
        out_specs=pl.BlockSpec((1, BLK), lambda j: (0, j)),
        out_shape=jax.ShapeDtypeStruct((1, NP), jnp.float32),
    )(cnt)


def _tc_final(part, t, dinv, c):
    return pl.pallas_call(
        _final_body,
        grid=(NP // BLK,),
        in_specs=[pl.BlockSpec((NT, BLK), lambda j: (0, j)),
                  pl.BlockSpec((1, BLK), lambda j: (0, j)),
                  pl.BlockSpec((1, BLK), lambda j: (0, j)),
                  _full((1, 1))],
        out_specs=pl.BlockSpec((1, BLK), lambda j: (0, j)),
        out_shape=jax.ShapeDtypeStruct((1, NP), jnp.float32),
    )(part, t, dinv, c)


def kernel(x, edge_index, W1l, b1, W1r, W2l, b2, W2r, W3l, b3, W3r, a, Wp, bp):
    src = edge_index[0]
    dst = edge_index[1]
    src2 = jnp.reshape(src, (NCH, K))
    dst2 = jnp.reshape(dst, (NCH, K))
    xp = jnp.pad(x, ((0, NP - N), (0, 0)))
    ones = jnp.ones((NP,), jnp.float32)
    a2 = jnp.reshape(a, (1, 1))
    b1r = jnp.reshape(b1, (1, D))
    b2r = jnp.reshape(b2, (1, D))
    uv = jnp.concatenate([Wp @ W3l, Wp @ W3r], axis=0)
    c = jnp.reshape(Wp @ b3 + bp, (1, 1))

    feat_segsum, scalar_segsum = _sc_kernels()
    cnt = scalar_segsum(ones, src, dst)
    dinv_r = _tc_dinv(cnt)
    dinv_c = jnp.reshape(dinv_r, (NP, 1))
    y1, z1 = _tc_l1(xp, W1l, W1r, b1r)
    agg1 = feat_segsum(y1, src2, dst2)
    y2, z2 = _tc_mid(agg1, z1, dinv_c, a2, W2l, W2r, b2r)
    agg2 = feat_segsum(y2, src2, dst2)
    st = _tc_head(agg2, z2, dinv_c, a2, uv)
    spart = scalar_segsum(st[:, 0], src, dst)
    tvec = jnp.reshape(st[:, 1], (1, NP))
    out = _tc_final(spart, tvec, dinv_r, c)
    return out[0, :N]


# speedup vs baseline: 1.2601x; 1.0263x over previous
import functools

import jax
import jax.numpy as jnp
from jax import lax
from jax.experimental import pallas as pl
from jax.experimental.pallas import tpu as pltpu
from jax.experimental.pallas import tpu_sc as plsc

N = 10000
E = 320000
NP = 10240
D = 128
BLK = 2048
NT = 32
NSUB = 16
K = 128
NCH = E // K
DH = D // 2
CPT = NCH // NSUB
CREM = NCH - CPT * NSUB
RPS = NP // NSUB
EPT = E // NT


def _feat_segsum_body(yab_hbm, src_hbm, dst_hbm, out_hbm,
                      acc, sidx, didx, rows0, rows1,
                      sem_g0, sem_g1, sem_s0, sem_s1):
    cid = lax.axis_index("c")
    sid = lax.axis_index("s")
    ytab = yab_hbm.at[cid]
    c0 = CPT * sid + jnp.minimum(sid, CREM)
    nch = CPT + (sid < CREM).astype(jnp.int32)

    pltpu.sync_copy(src_hbm.at[pl.ds(c0, CPT)], sidx.at[pl.ds(0, CPT)])
    pltpu.sync_copy(dst_hbm.at[pl.ds(c0, CPT)], didx.at[pl.ds(0, CPT)])

    @pl.when(sid < CREM)
    def _():
        pltpu.sync_copy(src_hbm.at[pl.ds(c0 + CPT, 1)], sidx.at[pl.ds(CPT, 1)])
        pltpu.sync_copy(dst_hbm.at[pl.ds(c0 + CPT, 1)], didx.at[pl.ds(CPT, 1)])

    zero = jnp.zeros((16,), jnp.float32)

    @plsc.parallel_loop(0, K, unroll=4)
    def _zrows(i):
        for j in range(DH // 16):
            rows0[i, pl.ds(j * 16, 16)] = zero

    for r in range(RPS // K):
        pltpu.sync_copy(rows0, acc.at[pl.ds(sid * RPS + r * K, K)])
    plsc.subcore_barrier()

    def _sc_wait(t, buf, ssem):
        pltpu.make_async_copy(buf, acc.at[didx.at[t]], ssem).wait()

    pltpu.make_async_copy(ytab.at[sidx.at[0]], rows0, sem_g0).start()

    def _step(t, buf, gsem, ssem, obuf, ogsem, ossem):
        @pl.when(t + 1 < nch)
        def _():
            @pl.when(t >= 1)
            def _():
                _sc_wait(t - 1, obuf, ossem)

            pltpu.make_async_copy(ytab.at[sidx.at[t + 1]], obuf, ogsem).start()

        pltpu.make_async_copy(ytab.at[sidx.at[t]], buf, gsem).wait()
        pltpu.make_async_copy(buf, acc.at[didx.at[t]], ssem).start(add=True)

    def mbody(t, carry):
        @pl.when(t % 2 == 0)
        def _():
            _step(t, rows0, sem_g0, sem_s0, rows1, sem_g1, sem_s1)

        @pl.when(t % 2 == 1)
        def _():
            _step(t, rows1, sem_g1, sem_s1, rows0, sem_g0, sem_s0)

        return carry

    lax.fori_loop(0, nch, mbody, 0)

    @pl.when(nch % 2 == 1)
    def _():
        _sc_wait(nch - 1, rows0, sem_s0)
        _sc_wait(nch - 2, rows1, sem_s1)

    @pl.when(nch % 2 == 0)
    def _():
        _sc_wait(nch - 1, rows1, sem_s1)
        _sc_wait(nch - 2, rows0, sem_s0)

    plsc.subcore_barrier()

    pltpu.sync_copy(acc.at[pl.ds(sid * RPS, RPS)],
                    out_hbm.at[cid, pl.ds(sid * RPS, RPS)])


def _scalar_segsum_body(tab_hbm, src_hbm, dst_hbm, out_hbm, tab, acc, sbuf, dbuf):
    wid = lax.axis_index("s") * 2 + lax.axis_index("c")
    base = wid * EPT
    pltpu.sync_copy(tab_hbm, tab)
    pltpu.sync_copy(src_hbm.at[pl.ds(base, EPT)], sbuf)
    pltpu.sync_copy(dst_hbm.at[pl.ds(base, EPT)], dbuf)

    zero = jnp.zeros((16,), jnp.float32)

    @plsc.parallel_loop(0, NP // 16, unroll=8)
    def _zero(i):
        acc[pl.ds(i * 16, 16)] = zero

    @plsc.parallel_loop(0, EPT // 16, unroll=8)
    def _groups(g):
        sv = sbuf[pl.ds(g * 16, 16)]
        dv = dbuf[pl.ds(g * 16, 16)]
        vals = plsc.load_gather(tab, [sv])
        plsc.addupdate_scatter(acc, [dv], vals)

    pltpu.sync_copy(acc, out_hbm.at[wid])


@functools.cache
def _sc_kernels():
    mesh = plsc.VectorSubcoreMesh(core_axis_name="c", subcore_axis_name="s")
    params = pltpu.CompilerParams(needs_layout_passes=False)
    feat = pl.kernel(
        _feat_segsum_body,
        mesh=mesh,
        compiler_params=pltpu.CompilerParams(
            needs_layout_passes=False, use_tc_tiling_on_sc=False),
        out_type=jax.ShapeDtypeStruct((2, NP, DH), jnp.float32),
        scratch_types=[
            pltpu.MemorySpace.VMEM_SHARED((NP, DH), jnp.float32),
            pltpu.VMEM((CPT + 1, K), jnp.int32),
            pltpu.VMEM((CPT + 1, K), jnp.int32),
            pltpu.VMEM((K, DH), jnp.float32),
            pltpu.VMEM((K, DH), jnp.float32),
            pltpu.SemaphoreType.DMA,
            pltpu.SemaphoreType.DMA,
            pltpu.SemaphoreType.DMA,
            pltpu.SemaphoreType.DMA,
        ],
    )
    scalar = pl.kernel(
        _scalar_segsum_body,
        mesh=mesh,
        compiler_params=params,
        out_type=jax.ShapeDtypeStruct((NT, NP), jnp.float32),
        scratch_types=[
            pltpu.VMEM((NP,), jnp.float32),
            pltpu.VMEM((NP,), jnp.float32),
            pltpu.VMEM((EPT,), jnp.int32),
            pltpu.VMEM((EPT,), jnp.int32),
        ],
    )
    return feat, scalar


_DN_T = (((1,), (1,)), ((), ()))


def _split_store(yab_ref, y):
    yab_ref[0] = y[:, :DH]
    yab_ref[1] = y[:, DH:]


def _l1_body(x_ref, cnt_ref, wl_ref, wr_ref, b_ref,
             yab_ref, z_ref, dinv_r_ref, dinv_c_ref):
    xb = x_ref[...]
    _split_store(yab_ref, lax.dot_general(xb, wl_ref[...], _DN_T,
                                          preferred_element_type=jnp.float32))
    z_ref[...] = lax.dot_general(xb, wr_ref[...], _DN_T,
                                 preferred_element_type=jnp.float32) + b_ref[...]
    s = jnp.sum(cnt_ref[...], axis=0, keepdims=True)
    dr = 1.0 / jnp.maximum(s, 1.0)
    dinv_r_ref[...] = dr
    dinv_c_ref[...] = jnp.transpose(dr, (1, 0))


def _mid_body(agg_ref, z_ref, dinv_ref, a_ref, wl_ref, wr_ref,
              b_ref, yab_ref, zo_ref):
    agg = jnp.concatenate([agg_ref[0], agg_ref[1]], axis=1)
    pre = agg * dinv_ref[...] + z_ref[...]
    h = jnp.where(pre >= 0, pre, a_ref[...] * pre)
    _split_store(yab_ref, lax.dot_general(h, wl_ref[...], _DN_T,
                                          preferred_element_type=jnp.float32))
    zo_ref[...] = lax.dot_general(h, wr_ref[...], _DN_T,
                                  preferred_element_type=jnp.float32) + b_ref[...]


def _head_body(agg_ref, z_ref, dinv_ref, a_ref, uv_ref, st_ref):
    agg = jnp.concatenate([agg_ref[0], agg_ref[1]], axis=1)
    pre = agg * dinv_ref[...] + z_ref[...]
    h = jnp.where(pre >= 0, pre, a_ref[...] * pre)
    st_ref[...] = lax.dot_general(uv_ref[...], h, _DN_T,
                                  preferred_element_type=jnp.float32)


def _final_body(part_ref, t_ref, dinv_ref, c_ref, out_ref):
    s = jnp.sum(part_ref[...], axis=0, keepdims=True)
    out_ref[...] = s * dinv_ref[...] + t_ref[...] + c_ref[...]


def _full(shape):
    return pl.BlockSpec(shape, lambda j: (0,) * len(shape))


def _nblk(shape2):
    return pl.BlockSpec(shape2, lambda j: (j, 0))


_AB_SPEC = pl.BlockSpec((2, BLK, DH), lambda j: (0, j, 0))
_AB_SHAPE = jax.ShapeDtypeStruct((2, NP, DH), jnp.float32)


def _tc_l1(x, cnt, wl, wr, b):
    return pl.pallas_call(
        _l1_body,
        grid=(NP // BLK,),
        in_specs=[_nblk((BLK, D)), pl.BlockSpec((NT, BLK), lambda j: (0, j)),
                  _full((D, D)), _full((D, D)), _full((1, D))],
        out_specs=[_AB_SPEC, _nblk((BLK, D)),
                   pl.BlockSpec((1, BLK), lambda j: (0, j)),
                   _nblk((BLK, 1))],
        out_shape=[_AB_SHAPE, jax.ShapeDtypeStruct((NP, D), jnp.float32),
                   jax.ShapeDtypeStruct((1, NP), jnp.float32),
                   jax.ShapeDtypeStruct((NP, 1), jnp.float32)],
    )(x, cnt, wl, wr, b)


def _tc_mid(agg, z, dinv, a, wl, wr, b):
    return pl.pallas_call(
        _mid_body,
        grid=(NP // BLK,),
        in_specs=[_AB_SPEC, _nblk((BLK, D)), _nblk((BLK, 1)),
                  _full((1, 1)), _full((D, D)), _full((D, D)), _full((1, D))],
        out_specs=[_AB_SPEC, _nblk((BLK, D))],
        out_shape=[_AB_SHAPE, jax.ShapeDtypeStruct((NP, D), jnp.float32)],
    )(agg, z, dinv, a, wl, wr, b)


def _tc_head(agg, z, dinv, a, uv):
    return pl.pallas_call(
        _head_body,
        grid=(NP // BLK,),
        in_specs=[_AB_SPEC, _nblk((BLK, D)), _nblk((BLK, 1)),
                  _full((1, 1)), _full((2, D))],
        out_specs=pl.BlockSpec((2, BLK), lambda j: (0, j)),
        out_shape=jax.ShapeDtypeStruct((2, NP), jnp.float32),
    )(agg, z, dinv, a, uv)


def _tc_final(part, t, dinv, c):
    return pl.pallas_call(
        _final_body,
        grid=(NP // BLK,),
        in_specs=[pl.BlockSpec((NT, BLK), lambda j: (0, j)),
                  pl.BlockSpec((1, BLK), lambda j: (0, j)),
                  pl.BlockSpec((1, BLK), lambda j: (0, j)),
                  _full((1, 1))],
        out_specs=pl.BlockSpec((1, BLK), lambda j: (0, j)),
        out_shape=jax.ShapeDtypeStruct((1, NP), jnp.float32),
    )(part, t, dinv, c)


def kernel(x, edge_index, W1l, b1, W1r, W2l, b2, W2r, W3l, b3, W3r, a, Wp, bp):
    src = edge_index[0]
    dst = edge_index[1]
    src2 = jnp.reshape(src, (NCH, K))
    dst2 = jnp.reshape(dst, (NCH, K))
    xp = jnp.pad(x, ((0, NP - N), (0, 0)))
    ones = jnp.ones((NP,), jnp.float32)
    a2 = jnp.reshape(a, (1, 1))
    b1r = jnp.reshape(b1, (1, D))
    b2r = jnp.reshape(b2, (1, D))
    uv = jnp.concatenate([Wp @ W3l, Wp @ W3r], axis=0)
    c = jnp.reshape(Wp @ b3 + bp, (1, 1))

    feat_segsum, scalar_segsum = _sc_kernels()
    cnt = scalar_segsum(ones, src, dst)
    y1, z1, dinv_r, dinv_c = _tc_l1(xp, cnt, W1l, W1r, b1r)
    agg1 = feat_segsum(y1, src2, dst2)
    y2, z2 = _tc_mid(agg1, z1, dinv_c, a2, W2l, W2r, b2r)
    agg2 = feat_segsum(y2, src2, dst2)
    st = _tc_head(agg2, z2, dinv_c, a2, uv)
    spart = scalar_segsum(st[0], src, dst)
    out = _tc_final(spart, st[1:2], dinv_r, c)
    return out[0, :N]


# speedup vs baseline: 1.2930x; 1.0261x over previous
import functools

import jax
import jax.numpy as jnp
from jax import lax
from jax.experimental import pallas as pl
from jax.experimental.pallas import tpu as pltpu
from jax.experimental.pallas import tpu_sc as plsc

N = 10000
E = 320000
NP = 10240
D = 128
BLK = 2048
NT = 32
NSUB = 16
K = 128
NCH = E // K
DH = D // 2
CPT = NCH // NSUB
CREM = NCH - CPT * NSUB
RPS = NP // NSUB
EPT = E // NT


def _feat_segsum_body(yab_hbm, src_hbm, dst_hbm, out_hbm, cnt_hbm,
                      acc, sidx, didx, rows0, rows1, cacc,
                      sem_g0, sem_g1, sem_s0, sem_s1):
    cid = lax.axis_index("c")
    sid = lax.axis_index("s")
    ytab = yab_hbm.at[cid]
    c0 = CPT * sid + jnp.minimum(sid, CREM)
    nch = CPT + (sid < CREM).astype(jnp.int32)

    pltpu.sync_copy(src_hbm.at[pl.ds(c0, CPT)], sidx.at[pl.ds(0, CPT)])
    pltpu.sync_copy(dst_hbm.at[pl.ds(c0, CPT)], didx.at[pl.ds(0, CPT)])

    @pl.when(sid < CREM)
    def _():
        pltpu.sync_copy(src_hbm.at[pl.ds(c0 + CPT, 1)], sidx.at[pl.ds(CPT, 1)])
        pltpu.sync_copy(dst_hbm.at[pl.ds(c0 + CPT, 1)], didx.at[pl.ds(CPT, 1)])

    zero = jnp.zeros((16,), jnp.float32)

    @plsc.parallel_loop(0, K, unroll=4)
    def _zrows(i):
        for j in range(DH // 16):
            rows0[i, pl.ds(j * 16, 16)] = zero

    @pl.when(cid == 0)
    def _():
        @plsc.parallel_loop(0, NP // 16, unroll=8)
        def _zcnt(i):
            cacc[pl.ds(i * 16, 16)] = zero

    for r in range(RPS // K):
        pltpu.sync_copy(rows0, acc.at[pl.ds(sid * RPS + r * K, K)])
    plsc.subcore_barrier()

    def _sc_wait(t, buf, ssem):
        pltpu.make_async_copy(buf, acc.at[didx.at[t]], ssem).wait()

    pltpu.make_async_copy(ytab.at[sidx.at[0]], rows0, sem_g0).start()

    def _step(t, buf, gsem, ssem, obuf, ogsem, ossem):
        @pl.when(t + 1 < nch)
        def _():
            @pl.when(t >= 1)
            def _():
                _sc_wait(t - 1, obuf, ossem)

            pltpu.make_async_copy(ytab.at[sidx.at[t + 1]], obuf, ogsem).start()

        pltpu.make_async_copy(ytab.at[sidx.at[t]], buf, gsem).wait()
        pltpu.make_async_copy(buf, acc.at[didx.at[t]], ssem).start(add=True)

    onesv = jnp.full((16,), 1.0, jnp.float32)

    def mbody(t, carry):
        @pl.when(cid == 0)
        def _():
            for g in range(K // 16):
                dv = didx[t, pl.ds(g * 16, 16)]
                plsc.addupdate_scatter(cacc, [dv], onesv)

        @pl.when(t % 2 == 0)
        def _():
            _step(t, rows0, sem_g0, sem_s0, rows1, sem_g1, sem_s1)

        @pl.when(t % 2 == 1)
        def _():
            _step(t, rows1, sem_g1, sem_s1, rows0, sem_g0, sem_s0)

        return carry

    lax.fori_loop(0, nch, mbody, 0)

    @pl.when(nch % 2 == 1)
    def _():
        _sc_wait(nch - 1, rows0, sem_s0)
        _sc_wait(nch - 2, rows1, sem_s1)

    @pl.when(nch % 2 == 0)
    def _():
        _sc_wait(nch - 1, rows1, sem_s1)
        _sc_wait(nch - 2, rows0, sem_s0)

    plsc.subcore_barrier()

    pltpu.sync_copy(acc.at[pl.ds(sid * RPS, RPS)],
                    out_hbm.at[cid, pl.ds(sid * RPS, RPS)])

    @pl.when(cid == 0)
    def _():
        pltpu.sync_copy(cacc, cnt_hbm.at[sid])


def _scalar_segsum_body(tab_hbm, src_hbm, dst_hbm, out_hbm, tab, acc, sbuf, dbuf):
    wid = lax.axis_index("s") * 2 + lax.axis_index("c")
    base = wid * EPT
    pltpu.sync_copy(tab_hbm, tab)
    pltpu.sync_copy(src_hbm.at[pl.ds(base, EPT)], sbuf)
    pltpu.sync_copy(dst_hbm.at[pl.ds(base, EPT)], dbuf)

    zero = jnp.zeros((16,), jnp.float32)

    @plsc.parallel_loop(0, NP // 16, unroll=8)
    def _zero(i):
        acc[pl.ds(i * 16, 16)] = zero

    @plsc.parallel_loop(0, EPT // 16, unroll=8)
    def _groups(g):
        sv = sbuf[pl.ds(g * 16, 16)]
        dv = dbuf[pl.ds(g * 16, 16)]
        vals = plsc.load_gather(tab, [sv])
        plsc.addupdate_scatter(acc, [dv], vals)

    pltpu.sync_copy(acc, out_hbm.at[wid])


@functools.cache
def _sc_kernels():
    mesh = plsc.VectorSubcoreMesh(core_axis_name="c", subcore_axis_name="s")
    params = pltpu.CompilerParams(needs_layout_passes=False)
    feat = pl.kernel(
        _feat_segsum_body,
        mesh=mesh,
        compiler_params=pltpu.CompilerParams(
            needs_layout_passes=False, use_tc_tiling_on_sc=False),
        out_type=[jax.ShapeDtypeStruct((2, NP, DH), jnp.float32),
                  jax.ShapeDtypeStruct((NSUB, NP), jnp.float32)],
        scratch_types=[
            pltpu.MemorySpace.VMEM_SHARED((NP, DH), jnp.float32),
            pltpu.VMEM((CPT + 1, K), jnp.int32),
            pltpu.VMEM((CPT + 1, K), jnp.int32),
            pltpu.VMEM((K, DH), jnp.float32),
            pltpu.VMEM((K, DH), jnp.float32),
            pltpu.VMEM((NP,), jnp.float32),
            pltpu.SemaphoreType.DMA,
            pltpu.SemaphoreType.DMA,
            pltpu.SemaphoreType.DMA,
            pltpu.SemaphoreType.DMA,
        ],
    )
    scalar = pl.kernel(
        _scalar_segsum_body,
        mesh=mesh,
        compiler_params=params,
        out_type=jax.ShapeDtypeStruct((NT, NP), jnp.float32),
        scratch_types=[
            pltpu.VMEM((NP,), jnp.float32),
            pltpu.VMEM((NP,), jnp.float32),
            pltpu.VMEM((EPT,), jnp.int32),
            pltpu.VMEM((EPT,), jnp.int32),
        ],
    )
    return feat, scalar


_DN_T = (((1,), (1,)), ((), ()))


def _split_store(yab_ref, y):
    yab_ref[0] = y[:, :DH]
    yab_ref[1] = y[:, DH:]


def _dinv_row(cnt_ref):
    s = jnp.sum(cnt_ref[...], axis=0, keepdims=True)
    return 1.0 / jnp.maximum(s, 1.0)


def _l1_body(x_ref, wl_ref, wr_ref, b_ref, yab_ref, z_ref):
    xb = x_ref[...]
    _split_store(yab_ref, lax.dot_general(xb, wl_ref[...], _DN_T,
                                          preferred_element_type=jnp.float32))
    z_ref[...] = lax.dot_general(xb, wr_ref[...], _DN_T,
                                 preferred_element_type=jnp.float32) + b_ref[...]


def _mid_body(agg_ref, z_ref, cnt_ref, a_ref, wl_ref, wr_ref,
              b_ref, yab_ref, zo_ref):
    agg = jnp.concatenate([agg_ref[0], agg_ref[1]], axis=1)
    dinv = jnp.transpose(_dinv_row(cnt_ref), (1, 0))
    pre = agg * dinv + z_ref[...]
    h = jnp.where(pre >= 0, pre, a_ref[...] * pre)
    _split_store(yab_ref, lax.dot_general(h, wl_ref[...], _DN_T,
                                          preferred_element_type=jnp.float32))
    zo_ref[...] = lax.dot_general(h, wr_ref[...], _DN_T,
                                  preferred_element_type=jnp.float32) + b_ref[...]


def _head_body(agg_ref, z_ref, cnt_ref, a_ref, uv_ref, st_ref):
    agg = jnp.concatenate([agg_ref[0], agg_ref[1]], axis=1)
    dinv = jnp.transpose(_dinv_row(cnt_ref), (1, 0))
    pre = agg * dinv + z_ref[...]
    h = jnp.where(pre >= 0, pre, a_ref[...] * pre)
    st_ref[...] = lax.dot_general(uv_ref[...], h, _DN_T,
                                  preferred_element_type=jnp.float32)


def _final_body(part_ref, t_ref, cnt_ref, c_ref, out_ref):
    s = jnp.sum(part_ref[...], axis=0, keepdims=True)
    out_ref[...] = s * _dinv_row(cnt_ref) + t_ref[...] + c_ref[...]


def _full(shape):
    return pl.BlockSpec(shape, lambda j: (0,) * len(shape))


def _nblk(shape2):
    return pl.BlockSpec(shape2, lambda j: (j, 0))


_AB_SPEC = pl.BlockSpec((2, BLK, DH), lambda j: (0, j, 0))
_AB_SHAPE = jax.ShapeDtypeStruct((2, NP, DH), jnp.float32)


_CNT_SPEC = pl.BlockSpec((NSUB, BLK), lambda j: (0, j))


def _tc_l1(x, wl, wr, b):
    return pl.pallas_call(
        _l1_body,
        grid=(NP // BLK,),
        in_specs=[_nblk((BLK, D)), _full((D, D)), _full((D, D)), _full((1, D))],
        out_specs=[_AB_SPEC, _nblk((BLK, D))],
        out_shape=[_AB_SHAPE, jax.ShapeDtypeStruct((NP, D), jnp.float32)],
    )(x, wl, wr, b)


def _tc_mid(agg, z, cnt, a, wl, wr, b):
    return pl.pallas_call(
        _mid_body,
        grid=(NP // BLK,),
        in_specs=[_AB_SPEC, _nblk((BLK, D)), _CNT_SPEC,
                  _full((1, 1)), _full((D, D)), _full((D, D)), _full((1, D))],
        out_specs=[_AB_SPEC, _nblk((BLK, D))],
        out_shape=[_AB_SHAPE, jax.ShapeDtypeStruct((NP, D), jnp.float32)],
    )(agg, z, cnt, a, wl, wr, b)


def _tc_head(agg, z, cnt, a, uv):
    return pl.pallas_call(
        _head_body,
        grid=(NP // BLK,),
        in_specs=[_AB_SPEC, _nblk((BLK, D)), _CNT_SPEC,
                  _full((1, 1)), _full((2, D))],
        out_specs=pl.BlockSpec((2, BLK), lambda j: (0, j)),
        out_shape=jax.ShapeDtypeStruct((2, NP), jnp.float32),
    )(agg, z, cnt, a, uv)


def _tc_final(part, t, cnt, c):
    return pl.pallas_call(
        _final_body,
        grid=(NP // BLK,),
        in_specs=[pl.BlockSpec((NT, BLK), lambda j: (0, j)),
                  pl.BlockSpec((1, BLK), lambda j: (0, j)),
                  _CNT_SPEC,
                  _full((1, 1))],
        out_specs=pl.BlockSpec((1, BLK), lambda j: (0, j)),
        out_shape=jax.ShapeDtypeStruct((1, NP), jnp.float32),
    )(part, t, cnt, c)


def kernel(x, edge_index, W1l, b1, W1r, W2l, b2, W2r, W3l, b3, W3r, a, Wp, bp):
    src = edge_index[0]
    dst = edge_index[1]
    src2 = jnp.reshape(src, (NCH, K))
    dst2 = jnp.reshape(dst, (NCH, K))
    xp = jnp.pad(x, ((0, NP - N), (0, 0)))
    a2 = jnp.reshape(a, (1, 1))
    b1r = jnp.reshape(b1, (1, D))
    b2r = jnp.reshape(b2, (1, D))
    uv = jnp.concatenate([Wp @ W3l, Wp @ W3r], axis=0)
    c = jnp.reshape(Wp @ b3 + bp, (1, 1))

    feat_segsum, scalar_segsum = _sc_kernels()
    y1, z1 = _tc_l1(xp, W1l, W1r, b1r)
    agg1, cnt = feat_segsum(y1, src2, dst2)
    y2, z2 = _tc_mid(agg1, z1, cnt, a2, W2l, W2r, b2r)
    agg2, _ = feat_segsum(y2, src2, dst2)
    st = _tc_head(agg2, z2, cnt, a2, uv)
    spart = scalar_segsum(st[0], src, dst)
    out = _tc_final(spart, st[1:2], cnt, c)
    return out[0, :N]


# speedup vs baseline: 1.5229x; 1.1778x over previous
import functools

import jax
import jax.numpy as jnp
from jax import lax
from jax.experimental import pallas as pl
from jax.experimental.pallas import tpu as pltpu
from jax.experimental.pallas import tpu_sc as plsc

N = 10000
E = 320000
NP = 10240
D = 128
BLK = 2048
NT = 32
NSUB = 16
K = 128
NCH = E // K
DH = D // 2
CPT = NCH // NSUB
CREM = NCH - CPT * NSUB
RPS = NP // NSUB
EPT = E // NT


def _feat_segsum_body(yab_hbm, src_hbm, dst_hbm, out_hbm, cnt_hbm,
                      acc, sidx, didx, rows0, rows1, rows2, cacc,
                      sem_g0, sem_g1, sem_g2, sem_s0, sem_s1, sem_s2):
    cid = lax.axis_index("c")
    sid = lax.axis_index("s")
    ytab = yab_hbm.at[cid]
    c0 = CPT * sid + jnp.minimum(sid, CREM)
    nch = CPT + (sid < CREM).astype(jnp.int32)

    pltpu.sync_copy(src_hbm.at[pl.ds(c0, CPT)], sidx.at[pl.ds(0, CPT)])
    pltpu.sync_copy(dst_hbm.at[pl.ds(c0, CPT)], didx.at[pl.ds(0, CPT)])

    @pl.when(sid < CREM)
    def _():
        pltpu.sync_copy(src_hbm.at[pl.ds(c0 + CPT, 1)], sidx.at[pl.ds(CPT, 1)])
        pltpu.sync_copy(dst_hbm.at[pl.ds(c0 + CPT, 1)], didx.at[pl.ds(CPT, 1)])

    zero = jnp.zeros((16,), jnp.float32)

    @plsc.parallel_loop(0, K, unroll=4)
    def _zrows(i):
        for j in range(DH // 16):
            rows0[i, pl.ds(j * 16, 16)] = zero

    @pl.when(cid == 0)
    def _():
        @plsc.parallel_loop(0, NP // 16, unroll=8)
        def _zcnt(i):
            cacc[pl.ds(i * 16, 16)] = zero

    for r in range(RPS // K):
        pltpu.sync_copy(rows0, acc.at[pl.ds(sid * RPS + r * K, K)])
    plsc.subcore_barrier()

    bufs = (rows0, rows1, rows2)
    gsems = (sem_g0, sem_g1, sem_g2)
    ssems = (sem_s0, sem_s1, sem_s2)

    def _sc_wait(t, b, ssem):
        pltpu.make_async_copy(bufs[b], acc.at[didx.at[t]], ssems[ssem]).wait()

    pltpu.make_async_copy(ytab.at[sidx.at[0]], rows0, sem_g0).start()
    pltpu.make_async_copy(ytab.at[sidx.at[1]], rows1, sem_g1).start()

    def _step(t, b):
        bn = (b + 2) % 3

        @pl.when(t + 2 < nch)
        def _():
            @pl.when(t >= 1)
            def _():
                _sc_wait(t - 1, bn, bn)

            pltpu.make_async_copy(ytab.at[sidx.at[t + 2]], bufs[bn],
                                  gsems[bn]).start()

        pltpu.make_async_copy(ytab.at[sidx.at[t]], bufs[b], gsems[b]).wait()
        pltpu.make_async_copy(bufs[b], acc.at[didx.at[t]],
                              ssems[b]).start(add=True)

    onesv = jnp.full((16,), 1.0, jnp.float32)

    def mbody(t, carry):
        @pl.when(cid == 0)
        def _():
            for g in range(K // 16):
                dv = didx[t, pl.ds(g * 16, 16)]
                plsc.addupdate_scatter(cacc, [dv], onesv)

        for b in range(3):
            @pl.when(t % 3 == b)
            def _(b=b):
                _step(t, b)

        return carry

    lax.fori_loop(0, nch, mbody, 0)

    for r in range(3):
        @pl.when(nch % 3 == r)
        def _(r=r):
            for q in (3, 2, 1):
                b = (r - q) % 3
                _sc_wait(nch - q, b, b)

    plsc.subcore_barrier()

    pltpu.sync_copy(acc.at[pl.ds(sid * RPS, RPS)],
                    out_hbm.at[cid, pl.ds(sid * RPS, RPS)])

    @pl.when(cid == 0)
    def _():
        pltpu.sync_copy(cacc, cnt_hbm.at[sid])


def _scalar_segsum_body(tab_hbm, src_hbm, dst_hbm, out_hbm, tab, acc, sbuf, dbuf):
    wid = lax.axis_index("s") * 2 + lax.axis_index("c")
    base = wid * EPT
    pltpu.sync_copy(tab_hbm, tab)
    pltpu.sync_copy(src_hbm.at[pl.ds(base, EPT)], sbuf)
    pltpu.sync_copy(dst_hbm.at[pl.ds(base, EPT)], dbuf)

    zero = jnp.zeros((16,), jnp.float32)

    @plsc.parallel_loop(0, NP // 16, unroll=8)
    def _zero(i):
        acc[pl.ds(i * 16, 16)] = zero

    @plsc.parallel_loop(0, EPT // 16, unroll=8)
    def _groups(g):
        sv = sbuf[pl.ds(g * 16, 16)]
        dv = dbuf[pl.ds(g * 16, 16)]
        vals = plsc.load_gather(tab, [sv])
        plsc.addupdate_scatter(acc, [dv], vals)

    pltpu.sync_copy(acc, out_hbm.at[wid])


@functools.cache
def _sc_kernels():
    mesh = plsc.VectorSubcoreMesh(core_axis_name="c", subcore_axis_name="s")
    params = pltpu.CompilerParams(needs_layout_passes=False)
    feat = pl.kernel(
        _feat_segsum_body,
        mesh=mesh,
        compiler_params=pltpu.CompilerParams(
            needs_layout_passes=False, use_tc_tiling_on_sc=False),
        out_type=[jax.ShapeDtypeStruct((2, NP, DH), jnp.float32),
                  jax.ShapeDtypeStruct((NSUB, NP), jnp.float32)],
        scratch_types=[
            pltpu.MemorySpace.VMEM_SHARED((NP, DH), jnp.float32),
            pltpu.VMEM((CPT + 1, K), jnp.int32),
            pltpu.VMEM((CPT + 1, K), jnp.int32),
            pltpu.VMEM((K, DH), jnp.float32),
            pltpu.VMEM((K, DH), jnp.float32),
            pltpu.VMEM((K, DH), jnp.float32),
            pltpu.VMEM((NP,), jnp.float32),
            pltpu.SemaphoreType.DMA,
            pltpu.SemaphoreType.DMA,
            pltpu.SemaphoreType.DMA,
            pltpu.SemaphoreType.DMA,
            pltpu.SemaphoreType.DMA,
            pltpu.SemaphoreType.DMA,
        ],
    )
    scalar = pl.kernel(
        _scalar_segsum_body,
        mesh=mesh,
        compiler_params=params,
        out_type=jax.ShapeDtypeStruct((NT, NP), jnp.float32),
        scratch_types=[
            pltpu.VMEM((NP,), jnp.float32),
            pltpu.VMEM((NP,), jnp.float32),
            pltpu.VMEM((EPT,), jnp.int32),
            pltpu.VMEM((EPT,), jnp.int32),
        ],
    )
    return feat, scalar


_DN_T = (((1,), (1,)), ((), ()))


def _split_store(yab_ref, y):
    yab_ref[0] = y[:, :DH]
    yab_ref[1] = y[:, DH:]


def _dinv_row(cnt_ref):
    s = jnp.sum(cnt_ref[...], axis=0, keepdims=True)
    return 1.0 / jnp.maximum(s, 1.0)


def _l1_body(x_ref, wl_ref, wr_ref, b_ref, yab_ref, z_ref):
    xb = x_ref[...]
    _split_store(yab_ref, lax.dot_general(xb, wl_ref[...], _DN_T,
                                          preferred_element_type=jnp.float32))
    z_ref[...] = lax.dot_general(xb, wr_ref[...], _DN_T,
                                 preferred_element_type=jnp.float32) + b_ref[...]


def _mid_body(agg_ref, z_ref, cnt_ref, a_ref, wl_ref, wr_ref,
              b_ref, yab_ref, zo_ref):
    agg = jnp.concatenate([agg_ref[0], agg_ref[1]], axis=1)
    dinv = jnp.transpose(_dinv_row(cnt_ref), (1, 0))
    pre = agg * dinv + z_ref[...]
    h = jnp.where(pre >= 0, pre, a_ref[...] * pre)
    _split_store(yab_ref, lax.dot_general(h, wl_ref[...], _DN_T,
                                          preferred_element_type=jnp.float32))
    zo_ref[...] = lax.dot_general(h, wr_ref[...], _DN_T,
                                  preferred_element_type=jnp.float32) + b_ref[...]


def _head_body(agg_ref, z_ref, cnt_ref, a_ref, uv_ref, st_ref):
    agg = jnp.concatenate([agg_ref[0], agg_ref[1]], axis=1)
    dinv = jnp.transpose(_dinv_row(cnt_ref), (1, 0))
    pre = agg * dinv + z_ref[...]
    h = jnp.where(pre >= 0, pre, a_ref[...] * pre)
    st_ref[...] = lax.dot_general(uv_ref[...], h, _DN_T,
                                  preferred_element_type=jnp.float32)


def _final_body(part_ref, t_ref, cnt_ref, c_ref, out_ref):
    s = jnp.sum(part_ref[...], axis=0, keepdims=True)
    out_ref[...] = s * _dinv_row(cnt_ref) + t_ref[...] + c_ref[...]


def _full(shape):
    return pl.BlockSpec(shape, lambda j: (0,) * len(shape))


def _nblk(shape2):
    return pl.BlockSpec(shape2, lambda j: (j, 0))


_AB_SPEC = pl.BlockSpec((2, BLK, DH), lambda j: (0, j, 0))
_AB_SHAPE = jax.ShapeDtypeStruct((2, NP, DH), jnp.float32)


_CNT_SPEC = pl.BlockSpec((NSUB, BLK), lambda j: (0, j))


def _tc_l1(x, wl, wr, b):
    return pl.pallas_call(
        _l1_body,
        grid=(NP // BLK,),
        in_specs=[_nblk((BLK, D)), _full((D, D)), _full((D, D)), _full((1, D))],
        out_specs=[_AB_SPEC, _nblk((BLK, D))],
        out_shape=[_AB_SHAPE, jax.ShapeDtypeStruct((NP, D), jnp.float32)],
    )(x, wl, wr, b)


def _tc_mid(agg, z, cnt, a, wl, wr, b):
    return pl.pallas_call(
        _mid_body,
        grid=(NP // BLK,),
        in_specs=[_AB_SPEC, _nblk((BLK, D)), _CNT_SPEC,
                  _full((1, 1)), _full((D, D)), _full((D, D)), _full((1, D))],
        out_specs=[_AB_SPEC, _nblk((BLK, D))],
        out_shape=[_AB_SHAPE, jax.ShapeDtypeStruct((NP, D), jnp.float32)],
    )(agg, z, cnt, a, wl, wr, b)


def _tc_head(agg, z, cnt, a, uv):
    return pl.pallas_call(
        _head_body,
        grid=(NP // BLK,),
        in_specs=[_AB_SPEC, _nblk((BLK, D)), _CNT_SPEC,
                  _full((1, 1)), _full((2, D))],
        out_specs=pl.BlockSpec((2, BLK), lambda j: (0, j)),
        out_shape=jax.ShapeDtypeStruct((2, NP), jnp.float32),
    )(agg, z, cnt, a, uv)


def _tc_final(part, t, cnt, c):
    return pl.pallas_call(
        _final_body,
        grid=(NP // BLK,),
        in_specs=[pl.BlockSpec((NT, BLK), lambda j: (0, j)),
                  pl.BlockSpec((1, BLK), lambda j: (0, j)),
                  _CNT_SPEC,
                  _full((1, 1))],
        out_specs=pl.BlockSpec((1, BLK), lambda j: (0, j)),
        out_shape=jax.ShapeDtypeStruct((1, NP), jnp.float32),
    )(part, t, cnt, c)


def kernel(x, edge_index, W1l, b1, W1r, W2l, b2, W2r, W3l, b3, W3r, a, Wp, bp):
    src = edge_index[0]
    dst = edge_index[1]
    src2 = jnp.reshape(src, (NCH, K))
    dst2 = jnp.reshape(dst, (NCH, K))
    xp = jnp.pad(x, ((0, NP - N), (0, 0)))
    a2 = jnp.reshape(a, (1, 1))
    b1r = jnp.reshape(b1, (1, D))
    b2r = jnp.reshape(b2, (1, D))
    uv = jnp.concatenate([Wp @ W3l, Wp @ W3r], axis=0)
    c = jnp.reshape(Wp @ b3 + bp, (1, 1))

    feat_segsum, scalar_segsum = _sc_kernels()
    y1, z1 = _tc_l1(xp, W1l, W1r, b1r)
    agg1, cnt = feat_segsum(y1, src2, dst2)
    y2, z2 = _tc_mid(agg1, z1, cnt, a2, W2l, W2r, b2r)
    agg2, _ = feat_segsum(y2, src2, dst2)
    st = _tc_head(agg2, z2, cnt, a2, uv)
    spart = scalar_segsum(st[0], src, dst)
    out = _tc_final(spart, st[1:2], cnt, c)
    return out[0, :N]


# speedup vs baseline: 1.5329x; 1.0066x over previous
import functools

import jax
import jax.numpy as jnp
from jax import lax
from jax.experimental import pallas as pl
from jax.experimental.pallas import tpu as pltpu
from jax.experimental.pallas import tpu_sc as plsc

N = 10000
E = 320000
NP = 10240
D = 128
BLK = 2048
NT = 32
NSUB = 16
K = 128
NCH = E // K
DH = D // 2
CPT = NCH // NSUB
CREM = NCH - CPT * NSUB
RPS = NP // NSUB
RB = 4
EPT = E // NT


def _feat_segsum_body(yab_hbm, src_hbm, dst_hbm, out_hbm, cnt_hbm,
                      acc, sidx, didx, rows0, rows1, rows2, rows3, cacc,
                      sem_g0, sem_g1, sem_g2, sem_g3,
                      sem_s0, sem_s1, sem_s2, sem_s3):
    cid = lax.axis_index("c")
    sid = lax.axis_index("s")
    ytab = yab_hbm.at[cid]
    c0 = CPT * sid + jnp.minimum(sid, CREM)
    nch = CPT + (sid < CREM).astype(jnp.int32)

    pltpu.sync_copy(src_hbm.at[pl.ds(c0, CPT)], sidx.at[pl.ds(0, CPT)])
    pltpu.sync_copy(dst_hbm.at[pl.ds(c0, CPT)], didx.at[pl.ds(0, CPT)])

    @pl.when(sid < CREM)
    def _():
        pltpu.sync_copy(src_hbm.at[pl.ds(c0 + CPT, 1)], sidx.at[pl.ds(CPT, 1)])
        pltpu.sync_copy(dst_hbm.at[pl.ds(c0 + CPT, 1)], didx.at[pl.ds(CPT, 1)])

    zero = jnp.zeros((16,), jnp.float32)

    @plsc.parallel_loop(0, K, unroll=4)
    def _zrows(i):
        for j in range(DH // 16):
            rows0[i, pl.ds(j * 16, 16)] = zero

    @pl.when(cid == 0)
    def _():
        @plsc.parallel_loop(0, NP // 16, unroll=8)
        def _zcnt(i):
            cacc[pl.ds(i * 16, 16)] = zero

    for r in range(RPS // K):
        pltpu.sync_copy(rows0, acc.at[pl.ds(sid * RPS + r * K, K)])
    plsc.subcore_barrier()

    bufs = (rows0, rows1, rows2, rows3)
    gsems = (sem_g0, sem_g1, sem_g2, sem_g3)
    ssems = (sem_s0, sem_s1, sem_s2, sem_s3)

    def _sc_wait(t, b):
        pltpu.make_async_copy(bufs[b], acc.at[didx.at[t]], ssems[b]).wait()

    for p in range(RB - 1):
        pltpu.make_async_copy(ytab.at[sidx.at[p]], bufs[p], gsems[p]).start()

    def _step(t, b):
        bn = (b + RB - 1) % RB

        @pl.when(t + RB - 1 < nch)
        def _():
            @pl.when(t >= 1)
            def _():
                _sc_wait(t - 1, bn)

            pltpu.make_async_copy(ytab.at[sidx.at[t + RB - 1]], bufs[bn],
                                  gsems[bn]).start()

        pltpu.make_async_copy(ytab.at[sidx.at[t]], bufs[b], gsems[b]).wait()
        pltpu.make_async_copy(bufs[b], acc.at[didx.at[t]],
                              ssems[b]).start(add=True)

    onesv = jnp.full((16,), 1.0, jnp.float32)

    def mbody(t, carry):
        @pl.when(cid == 0)
        def _():
            for g in range(K // 16):
                dv = didx[t, pl.ds(g * 16, 16)]
                plsc.addupdate_scatter(cacc, [dv], onesv)

        for b in range(RB):
            @pl.when(t % RB == b)
            def _(b=b):
                _step(t, b)

        return carry

    lax.fori_loop(0, nch, mbody, 0)

    for r in range(RB):
        @pl.when(nch % RB == r)
        def _(r=r):
            for q in range(RB, 0, -1):
                _sc_wait(nch - q, (r - q) % RB)

    plsc.subcore_barrier()

    pltpu.sync_copy(acc.at[pl.ds(sid * RPS, RPS)],
                    out_hbm.at[cid, pl.ds(sid * RPS, RPS)])

    @pl.when(cid == 0)
    def _():
        pltpu.sync_copy(cacc, cnt_hbm.at[sid])


def _scalar_segsum_body(tab_hbm, src_hbm, dst_hbm, out_hbm, tab, acc, sbuf, dbuf):
    wid = lax.axis_index("s") * 2 + lax.axis_index("c")
    base = wid * EPT
    pltpu.sync_copy(tab_hbm, tab)
    pltpu.sync_copy(src_hbm.at[pl.ds(base, EPT)], sbuf)
    pltpu.sync_copy(dst_hbm.at[pl.ds(base, EPT)], dbuf)

    zero = jnp.zeros((16,), jnp.float32)

    @plsc.parallel_loop(0, NP // 16, unroll=8)
    def _zero(i):
        acc[pl.ds(i * 16, 16)] = zero

    @plsc.parallel_loop(0, EPT // 16, unroll=8)
    def _groups(g):
        sv = sbuf[pl.ds(g * 16, 16)]
        dv = dbuf[pl.ds(g * 16, 16)]
        vals = plsc.load_gather(tab, [sv])
        plsc.addupdate_scatter(acc, [dv], vals)

    pltpu.sync_copy(acc, out_hbm.at[wid])


@functools.cache
def _sc_kernels():
    mesh = plsc.VectorSubcoreMesh(core_axis_name="c", subcore_axis_name="s")
    params = pltpu.CompilerParams(needs_layout_passes=False)
    feat = pl.kernel(
        _feat_segsum_body,
        mesh=mesh,
        compiler_params=pltpu.CompilerParams(
            needs_layout_passes=False, use_tc_tiling_on_sc=False),
        out_type=[jax.ShapeDtypeStruct((2, NP, DH), jnp.float32),
                  jax.ShapeDtypeStruct((NSUB, NP), jnp.float32)],
        scratch_types=[
            pltpu.MemorySpace.VMEM_SHARED((NP, DH), jnp.float32),
            pltpu.VMEM((CPT + 1, K), jnp.int32),
            pltpu.VMEM((CPT + 1, K), jnp.int32),
            pltpu.VMEM((K, DH), jnp.float32),
            pltpu.VMEM((K, DH), jnp.float32),
            pltpu.VMEM((K, DH), jnp.float32),
            pltpu.VMEM((K, DH), jnp.float32),
            pltpu.VMEM((NP,), jnp.float32),
            pltpu.SemaphoreType.DMA,
            pltpu.SemaphoreType.DMA,
            pltpu.SemaphoreType.DMA,
            pltpu.SemaphoreType.DMA,
            pltpu.SemaphoreType.DMA,
            pltpu.SemaphoreType.DMA,
            pltpu.SemaphoreType.DMA,
            pltpu.SemaphoreType.DMA,
        ],
    )
    scalar = pl.kernel(
        _scalar_segsum_body,
        mesh=mesh,
        compiler_params=params,
        out_type=jax.ShapeDtypeStruct((NT, NP), jnp.float32),
        scratch_types=[
            pltpu.VMEM((NP,), jnp.float32),
            pltpu.VMEM((NP,), jnp.float32),
            pltpu.VMEM((EPT,), jnp.int32),
            pltpu.VMEM((EPT,), jnp.int32),
        ],
    )
    return feat, scalar


_DN_T = (((1,), (1,)), ((), ()))


def _split_store(yab_ref, y):
    yab_ref[0] = y[:, :DH]
    yab_ref[1] = y[:, DH:]


def _dinv_row(cnt_ref):
    s = jnp.sum(cnt_ref[...], axis=0, keepdims=True)
    return 1.0 / jnp.maximum(s, 1.0)


def _l1_body(x_ref, wl_ref, wr_ref, b_ref, yab_ref, z_ref):
    xb = x_ref[...]
    _split_store(yab_ref, lax.dot_general(xb, wl_ref[...], _DN_T,
                                          preferred_element_type=jnp.float32))
    z_ref[...] = lax.dot_general(xb, wr_ref[...], _DN_T,
                                 preferred_element_type=jnp.float32) + b_ref[...]


def _mid_body(agg_ref, z_ref, cnt_ref, a_ref, wl_ref, wr_ref,
              b_ref, yab_ref, zo_ref):
    agg = jnp.concatenate([agg_ref[0], agg_ref[1]], axis=1)
    dinv = jnp.transpose(_dinv_row(cnt_ref), (1, 0))
    pre = agg * dinv + z_ref[...]
    h = jnp.where(pre >= 0, pre, a_ref[...] * pre)
    _split_store(yab_ref, lax.dot_general(h, wl_ref[...], _DN_T,
                                          preferred_element_type=jnp.float32))
    zo_ref[...] = lax.dot_general(h, wr_ref[...], _DN_T,
                                  preferred_element_type=jnp.float32) + b_ref[...]


def _head_body(agg_ref, z_ref, cnt_ref, a_ref, uv_ref, st_ref):
    agg = jnp.concatenate([agg_ref[0], agg_ref[1]], axis=1)
    dinv = jnp.transpose(_dinv_row(cnt_ref), (1, 0))
    pre = agg * dinv + z_ref[...]
    h = jnp.where(pre >= 0, pre, a_ref[...] * pre)
    st_ref[...] = lax.dot_general(uv_ref[...], h, _DN_T,
                                  preferred_element_type=jnp.float32)


def _final_body(part_ref, t_ref, cnt_ref, c_ref, out_ref):
    s = jnp.sum(part_ref[...], axis=0, keepdims=True)
    out_ref[...] = s * _dinv_row(cnt_ref) + t_ref[...] + c_ref[...]


def _full(shape):
    return pl.BlockSpec(shape, lambda j: (0,) * len(shape))


def _nblk(shape2):
    return pl.BlockSpec(shape2, lambda j: (j, 0))


_AB_SPEC = pl.BlockSpec((2, BLK, DH), lambda j: (0, j, 0))
_AB_SHAPE = jax.ShapeDtypeStruct((2, NP, DH), jnp.float32)


_CNT_SPEC = pl.BlockSpec((NSUB, BLK), lambda j: (0, j))


def _tc_l1(x, wl, wr, b):
    return pl.pallas_call(
        _l1_body,
        grid=(NP // BLK,),
        in_specs=[_nblk((BLK, D)), _full((D, D)), _full((D, D)), _full((1, D))],
        out_specs=[_AB_SPEC, _nblk((BLK, D))],
        out_shape=[_AB_SHAPE, jax.ShapeDtypeStruct((NP, D), jnp.float32)],
    )(x, wl, wr, b)


def _tc_mid(agg, z, cnt, a, wl, wr, b):
    return pl.pallas_call(
        _mid_body,
        grid=(NP // BLK,),
        in_specs=[_AB_SPEC, _nblk((BLK, D)), _CNT_SPEC,
                  _full((1, 1)), _full((D, D)), _full((D, D)), _full((1, D))],
        out_specs=[_AB_SPEC, _nblk((BLK, D))],
        out_shape=[_AB_SHAPE, jax.ShapeDtypeStruct((NP, D), jnp.float32)],
    )(agg, z, cnt, a, wl, wr, b)


def _tc_head(agg, z, cnt, a, uv):
    return pl.pallas_call(
        _head_body,
        grid=(NP // BLK,),
        in_specs=[_AB_SPEC, _nblk((BLK, D)), _CNT_SPEC,
                  _full((1, 1)), _full((2, D))],
        out_specs=pl.BlockSpec((2, BLK), lambda j: (0, j)),
        out_shape=jax.ShapeDtypeStruct((2, NP), jnp.float32),
    )(agg, z, cnt, a, uv)


def _tc_final(part, t, cnt, c):
    return pl.pallas_call(
        _final_body,
        grid=(NP // BLK,),
        in_specs=[pl.BlockSpec((NT, BLK), lambda j: (0, j)),
                  pl.BlockSpec((1, BLK), lambda j: (0, j)),
                  _CNT_SPEC,
                  _full((1, 1))],
        out_specs=pl.BlockSpec((1, BLK), lambda j: (0, j)),
        out_shape=jax.ShapeDtypeStruct((1, NP), jnp.float32),
    )(part, t, cnt, c)


def kernel(x, edge_index, W1l, b1, W1r, W2l, b2, W2r, W3l, b3, W3r, a, Wp, bp):
    src = edge_index[0]
    dst = edge_index[1]
    src2 = jnp.reshape(src, (NCH, K))
    dst2 = jnp.reshape(dst, (NCH, K))
    xp = jnp.pad(x, ((0, NP - N), (0, 0)))
    a2 = jnp.reshape(a, (1, 1))
    b1r = jnp.reshape(b1, (1, D))
    b2r = jnp.reshape(b2, (1, D))
    uv = jnp.concatenate([Wp @ W3l, Wp @ W3r], axis=0)
    c = jnp.reshape(Wp @ b3 + bp, (1, 1))

    feat_segsum, scalar_segsum = _sc_kernels()
    y1, z1 = _tc_l1(xp, W1l, W1r, b1r)
    agg1, cnt = feat_segsum(y1, src2, dst2)
    y2, z2 = _tc_mid(agg1, z1, cnt, a2, W2l, W2r, b2r)
    agg2, _ = feat_segsum(y2, src2, dst2)
    st = _tc_head(agg2, z2, cnt, a2, uv)
    spart = scalar_segsum(st[0], src, dst)
    out = _tc_final(spart, st[1:2], cnt, c)
    return out[0, :N]
